# baseline (speedup 1.0000x reference)
import jax
import jax.numpy as jnp
from jax.experimental import pallas as pl
from jax.experimental.pallas import tpu as pltpu


def _gru_block_kernel(x_ref, Wxz_ref, Wxr_ref, Wxh_ref, Whz_ref, Whr_ref,
                      Whh_ref, bxz_ref, bhz_ref, bxr_ref, bhr_ref, bxh_ref,
                      bhh_ref, Wout_ref, bout_ref, out_ref, h_ref, h_scr):
    _, B, F = x_ref.shape
    H = Whh_ref.shape[0]
    t = pl.program_id(1)
    nt = pl.num_programs(1)
    Wx = jnp.concatenate([Wxz_ref[...], Wxr_ref[...], Wxh_ref[...]], axis=1)
    bx = jnp.concatenate([bxz_ref[...] + bhz_ref[...],
                          bxr_ref[...] + bhr_ref[...],
                          bxh_ref[...] + bhh_ref[...]])
    xp = (jnp.dot(x_ref[0], Wx, preferred_element_type=jnp.float32) + bx)

    @pl.when(t == 0)
    def _():
        z = jax.nn.sigmoid(xp[:, :H])
        h_scr[...] = (1.0 - z) * jnp.tanh(xp[:, 2 * H:])

    @pl.when(t > 0)
    def _():
        Whzr = jnp.concatenate([Whz_ref[...], Whr_ref[...]], axis=1)
        h = h_scr[...]
        zr = jnp.dot(h, Whzr, preferred_element_type=jnp.float32)
        z = jax.nn.sigmoid(xp[:, :H] + zr[:, :H])
        r = jax.nn.sigmoid(xp[:, H:2 * H] + zr[:, H:])
        hc = jnp.dot(h * r, Whh_ref[...], preferred_element_type=jnp.float32)
        h_tilde = jnp.tanh(xp[:, 2 * H:] + hc)
        h_scr[...] = z * h + (1.0 - z) * h_tilde

    @pl.when(t == nt - 1)
    def _():
        h = h_scr[...]
        h_ref[...] = h
        out_ref[...] = (jnp.dot(h, Wout_ref[...],
                                preferred_element_type=jnp.float32)
                        + bout_ref[...])


def kernel(x, edge_index, edge_weight, W_l, b_l, W_r, b_r, att, b_gat,
           W_xz, b_xz, W_hz, b_hz, W_xr, b_xr, W_hr, b_hr, W_xh, b_xh,
           W_hh, b_hh, W_out, b_out):
    win, n, f = x.shape
    hid = W_hz.shape[0]
    block = 2000
    grid = (n // block, win)

    wspec = pl.BlockSpec((f, hid), lambda i, t: (0, 0))
    bspec = pl.BlockSpec((hid,), lambda i, t: (0,))
    out2d, h = pl.pallas_call(
        _gru_block_kernel,
        grid=grid,
        in_specs=[
            pl.BlockSpec((1, block, f), lambda i, t: (t, i, 0)),
            wspec, wspec, wspec, wspec, wspec, wspec,
            bspec, bspec, bspec, bspec, bspec, bspec,
            pl.BlockSpec((hid, 1), lambda i, t: (0, 0)),
            pl.BlockSpec((1,), lambda i, t: (0,)),
        ],
        out_specs=[
            pl.BlockSpec((block, 1), lambda i, t: (i, 0)),
            pl.BlockSpec((block, hid), lambda i, t: (i, 0)),
        ],
        out_shape=[
            jax.ShapeDtypeStruct((n, 1), jnp.float32),
            jax.ShapeDtypeStruct((n, hid), jnp.float32),
        ],
        scratch_shapes=[pltpu.VMEM((block, hid), jnp.float32)],
        compiler_params=pltpu.CompilerParams(
            dimension_semantics=("parallel", "arbitrary"),
        ),
    )(x, W_xz, W_xr, W_xh, W_hz, W_hr, W_hh,
      b_xz, b_hz, b_xr, b_hr, b_xh, b_hh, W_out, b_out)
    return out2d[:, 0], h


# speedup vs baseline: 1.5144x; 1.5144x over previous
import jax
import jax.numpy as jnp
from jax.experimental import pallas as pl
from jax.experimental.pallas import tpu as pltpu


def _gru_block_kernel(x_ref, Wxz_ref, Wxr_ref, Wxh_ref, Whz_ref, Whr_ref,
                      Whh_ref, bxz_ref, bhz_ref, bxr_ref, bhr_ref, bxh_ref,
                      bhh_ref, Wout_ref, bout_ref, out_ref, h_ref):
    win, B, F = x_ref.shape
    H = Whh_ref.shape[0]
    Wx = jnp.concatenate([Wxz_ref[...], Wxr_ref[...], Wxh_ref[...]], axis=1)
    Whzr = jnp.concatenate([Whz_ref[...], Whr_ref[...]], axis=1)
    bx = jnp.concatenate([bxz_ref[...] + bhz_ref[...],
                          bxr_ref[...] + bhr_ref[...],
                          bxh_ref[...] + bhh_ref[...]])
    xall = x_ref[...].reshape(win * B, F)
    xproj = (jnp.dot(xall, Wx, preferred_element_type=jnp.float32)
             + bx).reshape(win, B, 3 * H)
    nsplit = 2
    Bh = B // nsplit
    Whh = Whh_ref[...]
    hs = []
    for s in range(nsplit):
        z = jax.nn.sigmoid(xproj[0, s * Bh:(s + 1) * Bh, :H])
        hs.append((1.0 - z) * jnp.tanh(xproj[0, s * Bh:(s + 1) * Bh, 2 * H:]))
    for t in range(1, win):
        for s in range(nsplit):
            xp = xproj[t, s * Bh:(s + 1) * Bh]
            h = hs[s]
            zr = jnp.dot(h, Whzr, preferred_element_type=jnp.float32)
            z = jax.nn.sigmoid(xp[:, :H] + zr[:, :H])
            r = jax.nn.sigmoid(xp[:, H:2 * H] + zr[:, H:])
            hc = jnp.dot(h * r, Whh, preferred_element_type=jnp.float32)
            h_tilde = jnp.tanh(xp[:, 2 * H:] + hc)
            hs[s] = z * h + (1.0 - z) * h_tilde
    h = jnp.concatenate(hs, axis=0)
    h_ref[...] = h
    w_row = Wout_ref[...].reshape(1, H)
    out_ref[0] = (jax.lax.dot_general(
        w_row, h, (((1,), (1,)), ((), ())),
        preferred_element_type=jnp.float32) + bout_ref[...])


def kernel(x, edge_index, edge_weight, W_l, b_l, W_r, b_r, att, b_gat,
           W_xz, b_xz, W_hz, b_hz, W_xr, b_xr, W_hr, b_hr, W_xh, b_xh,
           W_hh, b_hh, W_out, b_out):
    win, n, f = x.shape
    hid = W_hz.shape[0]
    block = 2000
    grid = n // block

    wspec = pl.BlockSpec((f, hid), lambda i: (0, 0))
    bspec = pl.BlockSpec((hid,), lambda i: (0,))
    out2d, h = pl.pallas_call(
        _gru_block_kernel,
        grid=(grid,),
        in_specs=[
            pl.BlockSpec((win, block, f), lambda i: (0, i, 0)),
            wspec, wspec, wspec, wspec, wspec, wspec,
            bspec, bspec, bspec, bspec, bspec, bspec,
            pl.BlockSpec((hid, 1), lambda i: (0, 0)),
            pl.BlockSpec((1,), lambda i: (0,)),
        ],
        out_specs=[
            pl.BlockSpec((1, 1, block), lambda i: (i, 0, 0)),
            pl.BlockSpec((block, hid), lambda i: (i, 0)),
        ],
        out_shape=[
            jax.ShapeDtypeStruct((grid, 1, block), jnp.float32),
            jax.ShapeDtypeStruct((n, hid), jnp.float32),
        ],
        compiler_params=pltpu.CompilerParams(
            dimension_semantics=("parallel",),
        ),
    )(x, W_xz, W_xr, W_xh, W_hz, W_hr, W_hh,
      b_xz, b_hz, b_xr, b_hr, b_xh, b_hh, W_out, b_out)
    return out2d.reshape(n), h
